# baseline (speedup 1.0000x reference)
import functools

import jax
import jax.numpy as jnp
from jax import lax
from jax.experimental import pallas as pl
from jax.experimental.pallas import tpu as pltpu
from jax.experimental.pallas import tpu_sc as plsc

NUM_EXPERTS = 64
TOP_K = 8
N_GROUP = 8
TOPK_GROUP = 4
EXPERTS_PER_GROUP = NUM_EXPERTS // N_GROUP
ROUTED_SCALING_FACTOR = 2.5

NEG = float("-inf")
_T = 512


def _gate_body(x_ref, g_ref, b_ref, s_ref, sr_ref):
    logits = jnp.dot(
        x_ref[...].astype(jnp.bfloat16),
        g_ref[...].astype(jnp.bfloat16),
        preferred_element_type=jnp.float32,
    )
    s = jax.nn.sigmoid(logits)
    s_ref[...] = s
    sr_ref[...] = s + b_ref[...]


def _unflatten(x2, t):
    parts = [
        lax.slice(x2, (0, 8 * phi), (t // 16, 8 * phi + 8))
        for phi in range(16)
    ]
    return jnp.stack(parts, axis=1).reshape(t, TOP_K)


def _pack_body(iflat_ref, wflat_ref, idx_ref, w_ref):
    t = idx_ref.shape[0]
    idx_ref[...] = _unflatten(iflat_ref[...], t)
    w_ref[...] = _unflatten(wflat_ref[...], t)


def _tree(op, xs):
    xs = list(xs)
    while len(xs) > 1:
        nxt = [op(xs[i], xs[i + 1]) for i in range(0, len(xs) - 1, 2)]
        if len(xs) % 2:
            nxt.append(xs[-1])
        xs = nxt
    return xs[0]


def _route_body(sr_hbm, bias_hbm, idx_hbm, w_hbm, sr_v, bias_v, idx_v, w_v):
    info = plsc.get_sparse_core_info()
    nc, ns, lanes = info.num_cores, info.num_subcores, info.num_lanes
    nw = nc * ns
    n_tokens = sr_hbm.shape[0] // NUM_EXPERTS
    tpw = n_tokens // nw

    wid = lax.axis_index("s") * nc + lax.axis_index("c")
    base = wid * tpw
    pltpu.sync_copy(sr_hbm.at[pl.ds(base * NUM_EXPERTS, tpw * NUM_EXPERTS)], sr_v)
    pltpu.sync_copy(bias_hbm, bias_v)

    lane = lax.iota(jnp.int32, lanes)
    neg_v = jnp.full((lanes,), NEG, jnp.float32)

    @plsc.parallel_loop(0, tpw // lanes, unroll=2)
    def chunk(c):
        tok = c * lanes + lane
        row0 = tok * NUM_EXPERTS

        m1 = [None] * N_GROUP
        m2 = [None] * N_GROUP
        for g in range(N_GROUP):
            for j in range(EXPERTS_PER_GROUP):
                e = g * EXPERTS_PER_GROUP + j
                v = plsc.load_gather(sr_v, [row0 + e])
                if j == 0:
                    m1[g], m2[g] = v, neg_v
                else:
                    m2[g] = jnp.maximum(m2[g], jnp.minimum(m1[g], v))
                    m1[g] = jnp.maximum(m1[g], v)
        gs = [m1[g] + m2[g] for g in range(N_GROUP)]

        gmax = [None] * N_GROUP
        for g in range(N_GROUP):
            r = jnp.zeros((lanes,), jnp.int32)
            for h in range(N_GROUP):
                if h == g:
                    continue
                if h < g:
                    r = r + (gs[h] >= gs[g]).astype(jnp.int32)
                else:
                    r = r + (gs[h] > gs[g]).astype(jnp.int32)
            gmax[g] = jnp.where(r < TOPK_GROUP, m1[g], neg_v)

        idx_sel = [None] * TOP_K
        w_sel = [None] * TOP_K
        big = jnp.full((lanes,), 1 << 14, jnp.int32)
        for p in range(TOP_K):
            m = _tree(jnp.maximum, gmax)
            gstar = _tree(jnp.minimum, [
                jnp.where(gmax[g] == m, jnp.full((lanes,), g, jnp.int32), big)
                for g in range(N_GROUP)
            ])
            gstar = jnp.minimum(gstar, N_GROUP - 1)
            col0 = gstar * EXPERTS_PER_GROUP
            cols = [col0 + j for j in range(EXPERTS_PER_GROUP)]
            vals = [
                plsc.load_gather(sr_v, [row0 + cols[j]])
                for j in range(EXPERTS_PER_GROUP)
            ]
            estar = _tree(jnp.minimum, [
                jnp.where(vals[j] == m, cols[j], big)
                for j in range(EXPERTS_PER_GROUP)
            ])
            estar = jnp.minimum(estar, NUM_EXPERTS - 1)
            idx_sel[p] = estar
            w_sel[p] = m - plsc.load_gather(bias_v, [estar])
            plsc.store_scatter(sr_v, [row0 + estar], neg_v)
            newmax = _tree(jnp.maximum, [
                jnp.where(cols[j] == estar, neg_v, vals[j])
                for j in range(EXPERTS_PER_GROUP)
            ])
            for g in range(N_GROUP):
                hit = gstar == jnp.full((lanes,), g, jnp.int32)
                gmax[g] = jnp.where(hit, newmax, gmax[g])

        wsum = w_sel[0]
        for p in range(1, TOP_K):
            wsum = wsum + w_sel[p]
        scale = ROUTED_SCALING_FACTOR / (wsum + 1e-20)
        out0 = tok * TOP_K
        for p in range(TOP_K):
            plsc.store_scatter(idx_v, [out0 + p], idx_sel[p])
            plsc.store_scatter(w_v, [out0 + p], w_sel[p] * scale)

    pltpu.sync_copy(idx_v, idx_hbm.at[pl.ds(base * TOP_K, tpw * TOP_K)])
    pltpu.sync_copy(w_v, w_hbm.at[pl.ds(base * TOP_K, tpw * TOP_K)])


@jax.jit
def kernel(x, gate_kernel, e_score_correction_bias):
    n_tokens = x.shape[0]
    hidden = x.shape[1]
    grid = (n_tokens // _T,)
    bias2d = e_score_correction_bias.reshape(1, NUM_EXPERTS)
    scores, sr = pl.pallas_call(
        _gate_body,
        grid=grid,
        in_specs=[
            pl.BlockSpec((_T, hidden), lambda i: (i, 0)),
            pl.BlockSpec((hidden, NUM_EXPERTS), lambda i: (0, 0)),
            pl.BlockSpec((1, NUM_EXPERTS), lambda i: (0, 0)),
        ],
        out_specs=(
            pl.BlockSpec((_T, NUM_EXPERTS), lambda i: (i, 0)),
            pl.BlockSpec((_T, NUM_EXPERTS), lambda i: (i, 0)),
        ),
        out_shape=(
            jax.ShapeDtypeStruct((n_tokens, NUM_EXPERTS), jnp.float32),
            jax.ShapeDtypeStruct((n_tokens, NUM_EXPERTS), jnp.float32),
        ),
    )(x, gate_kernel, bias2d)

    info = plsc.get_sparse_core_info()
    nw = info.num_cores * info.num_subcores
    tpw = n_tokens // nw
    route = functools.partial(
        pl.kernel,
        mesh=plsc.VectorSubcoreMesh(core_axis_name="c", subcore_axis_name="s"),
        compiler_params=pltpu.CompilerParams(needs_layout_passes=False),
        out_type=(
            jax.ShapeDtypeStruct((n_tokens * TOP_K,), jnp.int32),
            jax.ShapeDtypeStruct((n_tokens * TOP_K,), jnp.float32),
        ),
        scratch_types=[
            pltpu.VMEM((tpw * NUM_EXPERTS,), jnp.float32),
            pltpu.VMEM((NUM_EXPERTS,), jnp.float32),
            pltpu.VMEM((tpw * TOP_K,), jnp.int32),
            pltpu.VMEM((tpw * TOP_K,), jnp.float32),
        ],
    )(_route_body)
    iflat, wflat = route(
        sr.reshape(n_tokens * NUM_EXPERTS), e_score_correction_bias)

    _TP = 2048
    i2 = iflat.reshape(n_tokens * TOP_K // 128, 128)
    w2 = wflat.reshape(n_tokens * TOP_K // 128, 128)
    idxs, ws = pl.pallas_call(
        _pack_body,
        grid=(n_tokens // _TP,),
        in_specs=[
            pl.BlockSpec((_TP * TOP_K // 128, 128), lambda i: (i, 0)),
            pl.BlockSpec((_TP * TOP_K // 128, 128), lambda i: (i, 0)),
        ],
        out_specs=(
            pl.BlockSpec((_TP, TOP_K), lambda i: (i, 0)),
            pl.BlockSpec((_TP, TOP_K), lambda i: (i, 0)),
        ),
        out_shape=(
            jax.ShapeDtypeStruct((n_tokens, TOP_K), jnp.int32),
            jax.ShapeDtypeStruct((n_tokens, TOP_K), jnp.float32),
        ),
    )(i2, w2)
    return (idxs, ws, scores)


# speedup vs baseline: 1.2399x; 1.2399x over previous
import functools

import jax
import jax.numpy as jnp
from jax import lax
from jax.experimental import pallas as pl
from jax.experimental.pallas import tpu as pltpu
from jax.experimental.pallas import tpu_sc as plsc

NUM_EXPERTS = 64
TOP_K = 8
N_GROUP = 8
TOPK_GROUP = 4
EXPERTS_PER_GROUP = NUM_EXPERTS // N_GROUP
ROUTED_SCALING_FACTOR = 2.5

NEG = float("-inf")
_T = 1024


def _gate_body(x_ref, g_ref, b_ref, s_ref, sr_ref):
    logits = jnp.dot(
        x_ref[...].astype(jnp.bfloat16),
        g_ref[...].astype(jnp.bfloat16),
        preferred_element_type=jnp.float32,
    )
    s = jax.nn.sigmoid(logits)
    s_ref[...] = s
    sr_ref[...] = s + b_ref[...]


def _tree(op, xs):
    xs = list(xs)
    while len(xs) > 1:
        nxt = [op(xs[i], xs[i + 1]) for i in range(0, len(xs) - 1, 2)]
        if len(xs) % 2:
            nxt.append(xs[-1])
        xs = nxt
    return xs[0]


def _route_body(sr_hbm, bias_hbm, idx_hbm, w_hbm, sr_v, bias_v, idx_v, w_v):
    info = plsc.get_sparse_core_info()
    nc, ns, lanes = info.num_cores, info.num_subcores, info.num_lanes
    nw = nc * ns
    n_tokens = sr_hbm.shape[0] // NUM_EXPERTS
    tpw = n_tokens // nw

    wid = lax.axis_index("s") * nc + lax.axis_index("c")
    base = wid * tpw
    pltpu.sync_copy(sr_hbm.at[pl.ds(base * NUM_EXPERTS, tpw * NUM_EXPERTS)], sr_v)
    pltpu.sync_copy(bias_hbm, bias_v)

    lane = lax.iota(jnp.int32, lanes)
    neg_v = jnp.full((lanes,), NEG, jnp.float32)

    @plsc.parallel_loop(0, tpw // lanes, unroll=4)
    def chunk(c):
        tok = c * lanes + lane
        row0 = tok * NUM_EXPERTS

        m1 = [None] * N_GROUP
        m2 = [None] * N_GROUP
        for g in range(N_GROUP):
            for j in range(EXPERTS_PER_GROUP):
                e = g * EXPERTS_PER_GROUP + j
                v = plsc.load_gather(sr_v, [row0 + e])
                if j == 0:
                    m1[g], m2[g] = v, neg_v
                else:
                    m2[g] = jnp.maximum(m2[g], jnp.minimum(m1[g], v))
                    m1[g] = jnp.maximum(m1[g], v)
        gs = [m1[g] + m2[g] for g in range(N_GROUP)]

        gmax = [None] * N_GROUP
        for g in range(N_GROUP):
            r = jnp.zeros((lanes,), jnp.int32)
            for h in range(N_GROUP):
                if h == g:
                    continue
                if h < g:
                    r = r + (gs[h] >= gs[g]).astype(jnp.int32)
                else:
                    r = r + (gs[h] > gs[g]).astype(jnp.int32)
            gmax[g] = jnp.where(r < TOPK_GROUP, m1[g], neg_v)

        idx_sel = [None] * TOP_K
        w_sel = [None] * TOP_K
        big = jnp.full((lanes,), 1 << 14, jnp.int32)
        for p in range(TOP_K):
            m = _tree(jnp.maximum, gmax)
            gstar = _tree(jnp.minimum, [
                jnp.where(gmax[g] == m, jnp.full((lanes,), g, jnp.int32), big)
                for g in range(N_GROUP)
            ])
            gstar = jnp.minimum(gstar, N_GROUP - 1)
            col0 = gstar * EXPERTS_PER_GROUP
            cols = [col0 + j for j in range(EXPERTS_PER_GROUP)]
            vals = [
                plsc.load_gather(sr_v, [row0 + cols[j]])
                for j in range(EXPERTS_PER_GROUP)
            ]
            estar = _tree(jnp.minimum, [
                jnp.where(vals[j] == m, cols[j], big)
                for j in range(EXPERTS_PER_GROUP)
            ])
            estar = jnp.minimum(estar, NUM_EXPERTS - 1)
            idx_sel[p] = estar
            w_sel[p] = m - plsc.load_gather(bias_v, [estar])
            plsc.store_scatter(sr_v, [row0 + estar], neg_v)
            newmax = _tree(jnp.maximum, [
                jnp.where(cols[j] == estar, neg_v, vals[j])
                for j in range(EXPERTS_PER_GROUP)
            ])
            for g in range(N_GROUP):
                hit = gstar == jnp.full((lanes,), g, jnp.int32)
                gmax[g] = jnp.where(hit, newmax, gmax[g])

        wsum = w_sel[0]
        for p in range(1, TOP_K):
            wsum = wsum + w_sel[p]
        scale = ROUTED_SCALING_FACTOR / (wsum + 1e-20)
        out0 = tok * TOP_K
        for p in range(TOP_K):
            plsc.store_scatter(idx_v, [out0 + p], idx_sel[p])
            plsc.store_scatter(w_v, [out0 + p], w_sel[p] * scale)

    pltpu.sync_copy(idx_v, idx_hbm.at[pl.ds(base * TOP_K, tpw * TOP_K)])
    pltpu.sync_copy(w_v, w_hbm.at[pl.ds(base * TOP_K, tpw * TOP_K)])


@jax.jit
def kernel(x, gate_kernel, e_score_correction_bias):
    n_tokens = x.shape[0]
    hidden = x.shape[1]
    grid = (n_tokens // _T,)
    bias2d = e_score_correction_bias.reshape(1, NUM_EXPERTS)
    scores, sr = pl.pallas_call(
        _gate_body,
        grid=grid,
        in_specs=[
            pl.BlockSpec((_T, hidden), lambda i: (i, 0)),
            pl.BlockSpec((hidden, NUM_EXPERTS), lambda i: (0, 0)),
            pl.BlockSpec((1, NUM_EXPERTS), lambda i: (0, 0)),
        ],
        out_specs=(
            pl.BlockSpec((_T, NUM_EXPERTS), lambda i: (i, 0)),
            pl.BlockSpec((_T, NUM_EXPERTS), lambda i: (i, 0)),
        ),
        out_shape=(
            jax.ShapeDtypeStruct((n_tokens, NUM_EXPERTS), jnp.float32),
            jax.ShapeDtypeStruct((n_tokens, NUM_EXPERTS), jnp.float32),
        ),
    )(x, gate_kernel, bias2d)

    info = plsc.get_sparse_core_info()
    nw = info.num_cores * info.num_subcores
    tpw = n_tokens // nw
    route = functools.partial(
        pl.kernel,
        mesh=plsc.VectorSubcoreMesh(core_axis_name="c", subcore_axis_name="s"),
        compiler_params=pltpu.CompilerParams(needs_layout_passes=False),
        out_type=(
            jax.ShapeDtypeStruct((n_tokens * TOP_K,), jnp.int32),
            jax.ShapeDtypeStruct((n_tokens * TOP_K,), jnp.float32),
        ),
        scratch_types=[
            pltpu.VMEM((tpw * NUM_EXPERTS,), jnp.float32),
            pltpu.VMEM((NUM_EXPERTS,), jnp.float32),
            pltpu.VMEM((tpw * TOP_K,), jnp.int32),
            pltpu.VMEM((tpw * TOP_K,), jnp.float32),
        ],
    )(_route_body)
    idxs, ws = route(sr.reshape(n_tokens * NUM_EXPERTS), e_score_correction_bias)
    return (
        idxs.reshape(n_tokens, TOP_K),
        ws.reshape(n_tokens, TOP_K),
        scores,
    )
